# baseline (speedup 1.0000x reference)
import jax
import jax.numpy as jnp
from jax import lax
from jax.experimental import pallas as pl
from jax.experimental.pallas import tpu as pltpu

_NCHUNK = 4


def _ffe_body(ens_ref, w_ref, x_ref, k0_hbm, k1_hbm, o_ref,
              k0_buf, k1_buf, sems):
    BK_rows, S = ens_ref.shape
    E, D, BKd = k1_buf.shape
    H = E * BKd
    T = x_ref.shape[0]
    B = T // S
    K = BK_rows // B
    EG = E // _NCHUNK
    RG = H // _NCHUNK

    for g in range(_NCHUNK):
        pltpu.make_async_copy(
            k0_hbm.at[pl.ds(g * RG, RG), :],
            k0_buf.at[pl.ds(g * RG, RG), :],
            sems.at[2 * g]).start()
        pltpu.make_async_copy(
            k1_hbm.at[pl.ds(g * EG, EG)],
            k1_buf.at[pl.ds(g * EG, EG)],
            sems.at[2 * g + 1]).start()

    iota_e = lax.broadcasted_iota(jnp.int32, (E, 1), 0)
    cols = []
    for b in range(B):
        ct = jnp.zeros((E, S), jnp.float32)
        for k in range(K):
            row = b * K + k
            ct = ct + jnp.where(ens_ref[row][None, :] == iota_e,
                                w_ref[row][None, :], 0.0)
        cols.append(ct)
    cT = jnp.concatenate(cols, axis=1)

    blk = lax.broadcasted_iota(jnp.int32, (E, H), 1) // BKd
    expand = jnp.where(lax.broadcasted_iota(jnp.int32, (E, H), 0) == blk,
                       1.0, 0.0)
    scale = jax.lax.dot_general(cT, expand, (((0,), (0,)), ((), ())),
                                preferred_element_type=jnp.float32)

    y = jnp.zeros((T, D), jnp.float32)
    for g in range(_NCHUNK):
        pltpu.make_async_copy(
            k0_hbm.at[pl.ds(g * RG, RG), :],
            k0_buf.at[pl.ds(g * RG, RG), :],
            sems.at[2 * g]).wait()
        h = jax.lax.dot_general(x_ref[...], k0_buf[pl.ds(g * RG, RG), :],
                                (((1,), (1,)), ((), ())),
                                preferred_element_type=jnp.float32)
        h = jnp.maximum(h, 0.0) * scale[:, g * RG:(g + 1) * RG]
        pltpu.make_async_copy(
            k1_hbm.at[pl.ds(g * EG, EG)],
            k1_buf.at[pl.ds(g * EG, EG)],
            sems.at[2 * g + 1]).wait()
        k1t = jnp.transpose(k1_buf[pl.ds(g * EG, EG)],
                            (0, 2, 1)).reshape(RG, D)
        y = y + jax.lax.dot_general(h, k1t, (((1,), (0,)), ((), ())),
                                    preferred_element_type=jnp.float32)
    o_ref[...] = y


def kernel(x, weights, ensembles, kernels_0, kernels_1):
    B, S, D = x.shape
    E, BK, _ = kernels_0.shape
    _, K, _ = weights.shape
    T = B * S

    x2 = x.reshape(T, D)
    ens2 = ensembles.astype(jnp.int32).reshape(B * K, S)
    w2 = weights.reshape(B * K, S)
    k0r = kernels_0.reshape(E * BK, D)

    out = pl.pallas_call(
        _ffe_body,
        in_specs=[
            pl.BlockSpec((B * K, S), lambda: (0, 0)),
            pl.BlockSpec((B * K, S), lambda: (0, 0)),
            pl.BlockSpec((T, D), lambda: (0, 0)),
            pl.BlockSpec(memory_space=pltpu.MemorySpace.HBM),
            pl.BlockSpec(memory_space=pltpu.MemorySpace.HBM),
        ],
        out_specs=pl.BlockSpec((T, D), lambda: (0, 0)),
        out_shape=jax.ShapeDtypeStruct((T, D), jnp.float32),
        scratch_shapes=[
            pltpu.VMEM((E * BK, D), jnp.float32),
            pltpu.VMEM((E, D, BK), jnp.float32),
            pltpu.SemaphoreType.DMA((2 * _NCHUNK,)),
        ],
    )(ens2, w2, x2, k0r, kernels_1)

    return out.reshape(B, S, D)


# speedup vs baseline: 1.0528x; 1.0528x over previous
import jax
import jax.numpy as jnp
from jax import lax
from jax.experimental import pallas as pl
from jax.experimental.pallas import tpu as pltpu

_NCHUNK = 1


def _ffe_body(ens_ref, w_ref, x_ref, k0_hbm, k1_hbm, o_ref,
              k0_buf, k1_buf, sems):
    BK_rows, S = ens_ref.shape
    E, D, BKd = k1_buf.shape
    H = E * BKd
    T = x_ref.shape[0]
    B = T // S
    K = BK_rows // B
    EG = E // _NCHUNK
    RG = H // _NCHUNK

    for g in range(_NCHUNK):
        pltpu.make_async_copy(
            k0_hbm.at[pl.ds(g * RG, RG), :],
            k0_buf.at[pl.ds(g * RG, RG), :],
            sems.at[2 * g]).start()
        pltpu.make_async_copy(
            k1_hbm.at[pl.ds(g * EG, EG)],
            k1_buf.at[pl.ds(g * EG, EG)],
            sems.at[2 * g + 1]).start()

    iota_e = lax.broadcasted_iota(jnp.int32, (E, 1), 0)
    cols = []
    for b in range(B):
        ct = jnp.zeros((E, S), jnp.float32)
        for k in range(K):
            row = b * K + k
            ct = ct + jnp.where(ens_ref[row][None, :] == iota_e,
                                w_ref[row][None, :], 0.0)
        cols.append(ct)
    cT = jnp.concatenate(cols, axis=1)

    blk = lax.broadcasted_iota(jnp.int32, (E, H), 1) // BKd
    expand = jnp.where(lax.broadcasted_iota(jnp.int32, (E, H), 0) == blk,
                       1.0, 0.0)
    scale = jax.lax.dot_general(cT, expand, (((0,), (0,)), ((), ())),
                                preferred_element_type=jnp.float32)

    y = jnp.zeros((T, D), jnp.float32)
    for g in range(_NCHUNK):
        pltpu.make_async_copy(
            k0_hbm.at[pl.ds(g * RG, RG), :],
            k0_buf.at[pl.ds(g * RG, RG), :],
            sems.at[2 * g]).wait()
        h = jax.lax.dot_general(x_ref[...], k0_buf[pl.ds(g * RG, RG), :],
                                (((1,), (1,)), ((), ())),
                                preferred_element_type=jnp.float32)
        h = jnp.maximum(h, 0.0) * scale[:, g * RG:(g + 1) * RG]
        pltpu.make_async_copy(
            k1_hbm.at[pl.ds(g * EG, EG)],
            k1_buf.at[pl.ds(g * EG, EG)],
            sems.at[2 * g + 1]).wait()
        k1t = jnp.transpose(k1_buf[pl.ds(g * EG, EG)],
                            (0, 2, 1)).reshape(RG, D)
        y = y + jax.lax.dot_general(h, k1t, (((1,), (0,)), ((), ())),
                                    preferred_element_type=jnp.float32)
    o_ref[...] = y


def kernel(x, weights, ensembles, kernels_0, kernels_1):
    B, S, D = x.shape
    E, BK, _ = kernels_0.shape
    _, K, _ = weights.shape
    T = B * S

    x2 = x.reshape(T, D)
    ens2 = ensembles.astype(jnp.int32).reshape(B * K, S)
    w2 = weights.reshape(B * K, S)
    k0r = kernels_0.reshape(E * BK, D)

    out = pl.pallas_call(
        _ffe_body,
        in_specs=[
            pl.BlockSpec((B * K, S), lambda: (0, 0)),
            pl.BlockSpec((B * K, S), lambda: (0, 0)),
            pl.BlockSpec((T, D), lambda: (0, 0)),
            pl.BlockSpec(memory_space=pltpu.MemorySpace.HBM),
            pl.BlockSpec(memory_space=pltpu.MemorySpace.HBM),
        ],
        out_specs=pl.BlockSpec((T, D), lambda: (0, 0)),
        out_shape=jax.ShapeDtypeStruct((T, D), jnp.float32),
        scratch_shapes=[
            pltpu.VMEM((E * BK, D), jnp.float32),
            pltpu.VMEM((E, D, BK), jnp.float32),
            pltpu.SemaphoreType.DMA((2 * _NCHUNK,)),
        ],
    )(ens2, w2, x2, k0r, kernels_1)

    return out.reshape(B, S, D)
